# baseline (speedup 1.0000x reference)
import functools
import math

import jax
import jax.numpy as jnp
from jax import lax
from jax.experimental import pallas as pl
from jax.experimental.pallas import tpu as pltpu
from jax.experimental.pallas import tpu_sc as plsc

_D = 128
_LANES = 16
_NC, _NS = 2, 16
_NW = _NC * _NS
_SCALE = math.sqrt(_D)
_CH = 128


def _make_kernel(batch: int):
    b_per_w = batch // _NW
    n_chunks = b_per_w // _CH
    n_pairs = n_chunks // 2
    assert n_chunks % 2 == 0 and n_pairs >= 2

    mesh = plsc.VectorSubcoreMesh(
        core_axis_name="c", subcore_axis_name="s",
        num_cores=_NC, num_subcores=_NS,
    )

    @functools.partial(
        pl.kernel,
        out_type=jax.ShapeDtypeStruct((batch, _D), jnp.float32),
        mesh=mesh,
        scratch_types=[
            pltpu.VMEM((b_per_w,), jnp.int32),
            pltpu.VMEM((_CH, _D), jnp.float32),
            pltpu.VMEM((_CH, _D), jnp.float32),
            pltpu.VMEM((_CH, _D), jnp.float32),
            pltpu.VMEM((_CH, _D), jnp.float32),
            pltpu.SemaphoreType.DMA,
            pltpu.SemaphoreType.DMA,
            pltpu.SemaphoreType.DMA,
            pltpu.SemaphoreType.DMA,
        ],
    )
    def emb(idx_hbm, table_hbm, out_hbm, idx_v,
            g0, g1, w0, w1, gs0, gs1, ws0, ws1):
        wid = lax.axis_index("s") * _NC + lax.axis_index("c")
        base = wid * b_per_w
        pltpu.sync_copy(idx_hbm.at[pl.ds(base, b_per_w)], idx_v)

        gbuf = (g0, g1)
        wbuf = (w0, w1)
        gsem = (gs0, gs1)
        wsem = (ws0, ws1)

        def start_gather(c, b):
            pltpu.async_copy(
                table_hbm.at[idx_v.at[pl.ds(c * _CH, _CH)]], gbuf[b], gsem[b])

        def wait_gather(b):
            pltpu.make_async_copy(
                table_hbm.at[idx_v.at[pl.ds(0, _CH)]], gbuf[b], gsem[b]).wait()

        def start_write(c, b):
            pltpu.async_copy(
                wbuf[b], out_hbm.at[pl.ds(base + c * _CH, _CH)], wsem[b])

        def wait_write(b):
            pltpu.make_async_copy(
                wbuf[b], out_hbm.at[pl.ds(0, _CH)], wsem[b]).wait()

        def scale(b):
            g, w = gbuf[b], wbuf[b]

            @plsc.parallel_loop(0, _CH, step=1, unroll=4)
            def _do_row(r):
                for j in range(_D // _LANES):
                    sl = pl.ds(j * _LANES, _LANES)
                    w[r, sl] = g[r, sl] * _SCALE

        start_gather(0, 0)
        start_gather(1, 1)
        for b in range(2):
            wait_gather(b)
            scale(b)
            start_write(b, b)
            start_gather(b + 2, b)

        def pair_body(p, carry):
            c = 2 * p
            for b in range(2):
                wait_gather(b)
                wait_write(b)
                scale(b)
                start_write(c + b, b)
                start_gather(c + b + 2, b)
            return carry

        lax.fori_loop(1, n_pairs - 1, pair_body, 0)

        c = n_chunks - 2
        for b in range(2):
            wait_gather(b)
            wait_write(b)
            scale(b)
            start_write(c + b, b)
        for b in range(2):
            wait_write(b)

    return emb


def kernel(x, lookup_table):
    batch, seq = x.shape
    idx = x.reshape(batch * seq).astype(jnp.int32)
    out = _make_kernel(batch * seq)(idx, lookup_table)
    return out.reshape(batch, seq, _D)


# speedup vs baseline: 1.0089x; 1.0089x over previous
import functools
import math

import jax
import jax.numpy as jnp
from jax import lax
from jax.experimental import pallas as pl
from jax.experimental.pallas import tpu as pltpu
from jax.experimental.pallas import tpu_sc as plsc

_D = 128
_LANES = 16
_NC, _NS = 2, 16
_NW = _NC * _NS
_SCALE = math.sqrt(_D)
_CH = 200
_NBUF = 4


def _make_kernel(batch: int):
    b_per_w = batch // _NW
    n_chunks = b_per_w // _CH
    assert b_per_w % _CH == 0 and (n_chunks - 4) % _NBUF == 0

    mesh = plsc.VectorSubcoreMesh(
        core_axis_name="c", subcore_axis_name="s",
        num_cores=_NC, num_subcores=_NS,
    )

    @functools.partial(
        pl.kernel,
        out_type=jax.ShapeDtypeStruct((batch, _D), jnp.float32),
        mesh=mesh,
        scratch_types=[
            pltpu.VMEM((b_per_w,), jnp.int32),
            pltpu.VMEM((_CH, _D), jnp.float32),
            pltpu.VMEM((_CH, _D), jnp.float32),
            pltpu.VMEM((_CH, _D), jnp.float32),
            pltpu.VMEM((_CH, _D), jnp.float32),
            pltpu.SemaphoreType.DMA,
            pltpu.SemaphoreType.DMA,
            pltpu.SemaphoreType.DMA,
            pltpu.SemaphoreType.DMA,
            pltpu.SemaphoreType.DMA,
            pltpu.SemaphoreType.DMA,
            pltpu.SemaphoreType.DMA,
            pltpu.SemaphoreType.DMA,
        ],
    )
    def emb(idx_hbm, table_hbm, out_hbm, idx_v,
            b0, b1, b2, b3, gs0, gs1, gs2, gs3, ws0, ws1, ws2, ws3):
        wid = lax.axis_index("s") * _NC + lax.axis_index("c")
        base = wid * b_per_w
        pltpu.sync_copy(idx_hbm.at[pl.ds(base, b_per_w)], idx_v)

        buf = (b0, b1, b2, b3)
        gsem = (gs0, gs1, gs2, gs3)
        wsem = (ws0, ws1, ws2, ws3)

        def start_gather(c, b):
            off = c * _CH
            pltpu.async_copy(
                table_hbm.at[idx_v.at[pl.ds(off, 128)]],
                buf[b].at[pl.ds(0, 128)], gsem[b])
            pltpu.async_copy(
                table_hbm.at[idx_v.at[pl.ds(off + 128, _CH - 128)]],
                buf[b].at[pl.ds(128, _CH - 128)], gsem[b])

        def wait_gather(b):
            pltpu.make_async_copy(
                table_hbm.at[idx_v.at[pl.ds(0, _CH)]], buf[b], gsem[b]).wait()

        def start_write(c, b):
            pltpu.async_copy(
                buf[b], out_hbm.at[pl.ds(base + c * _CH, _CH)], wsem[b])

        def wait_write(b):
            pltpu.make_async_copy(
                buf[b], out_hbm.at[pl.ds(0, _CH)], wsem[b]).wait()

        def scale(b):
            g = buf[b]

            @plsc.parallel_loop(0, _CH, step=1, unroll=4)
            def _do_row(r):
                for j in range(_D // _LANES):
                    sl = pl.ds(j * _LANES, _LANES)
                    g[r, sl] = g[r, sl] * _SCALE

        start_gather(0, 0)
        start_gather(1, 1)
        for c in range(2):
            wait_gather(c)
            scale(c)
            start_write(c, c)
            start_gather(c + 2, c + 2)

        def quad_body(p, carry):
            c0 = 2 + p * _NBUF
            for k in range(_NBUF):
                b = (2 + k) % _NBUF
                c = c0 + k
                wait_gather(b)
                scale(b)
                start_write(c, b)
                wait_write((b + 2) % _NBUF)
                start_gather(c + 2, (b + 2) % _NBUF)
            return carry

        lax.fori_loop(0, (n_chunks - 4) // _NBUF, quad_body, 0)

        for c in range(n_chunks - 2, n_chunks):
            b = c % _NBUF
            wait_gather(b)
            scale(b)
            start_write(c, b)
        for b in range(_NBUF):
            wait_write(b)

    return emb


def kernel(x, lookup_table):
    batch, seq = x.shape
    idx = x.reshape(batch * seq).astype(jnp.int32)
    out = _make_kernel(batch * seq)(idx, lookup_table)
    return out.reshape(batch, seq, _D)


# speedup vs baseline: 1.0114x; 1.0024x over previous
import functools
import math

import jax
import jax.numpy as jnp
from jax import lax
from jax.experimental import pallas as pl
from jax.experimental.pallas import tpu as pltpu
from jax.experimental.pallas import tpu_sc as plsc

_D = 128
_LANES = 16
_NC, _NS = 2, 16
_NW = _NC * _NS
_SCALE = math.sqrt(_D)
_NBUF = 4


def _make_kernel(batch: int, seq: int):
    rows_per_w = batch // _NW
    n_chunks = rows_per_w
    assert batch % _NW == 0 and (n_chunks - 4) % _NBUF == 0
    g0_len = min(seq, 128)
    g1_len = seq - g0_len

    mesh = plsc.VectorSubcoreMesh(
        core_axis_name="c", subcore_axis_name="s",
        num_cores=_NC, num_subcores=_NS,
    )

    @functools.partial(
        pl.kernel,
        out_type=jax.ShapeDtypeStruct((batch, seq, _D), jnp.float32),
        mesh=mesh,
        scratch_types=[
            pltpu.VMEM((rows_per_w, seq), jnp.int32),
            pltpu.VMEM((seq, _D), jnp.float32),
            pltpu.VMEM((seq, _D), jnp.float32),
            pltpu.VMEM((seq, _D), jnp.float32),
            pltpu.VMEM((seq, _D), jnp.float32),
            pltpu.SemaphoreType.DMA,
            pltpu.SemaphoreType.DMA,
            pltpu.SemaphoreType.DMA,
            pltpu.SemaphoreType.DMA,
            pltpu.SemaphoreType.DMA,
            pltpu.SemaphoreType.DMA,
            pltpu.SemaphoreType.DMA,
            pltpu.SemaphoreType.DMA,
        ],
    )
    def emb(idx_hbm, table_hbm, out_hbm, idx_v,
            b0, b1, b2, b3, gs0, gs1, gs2, gs3, ws0, ws1, ws2, ws3):
        wid = lax.axis_index("s") * _NC + lax.axis_index("c")
        base = wid * rows_per_w
        pltpu.sync_copy(idx_hbm.at[pl.ds(base, rows_per_w)], idx_v)

        buf = (b0, b1, b2, b3)
        gsem = (gs0, gs1, gs2, gs3)
        wsem = (ws0, ws1, ws2, ws3)

        def start_gather(c, b):
            pltpu.async_copy(
                table_hbm.at[idx_v.at[c, pl.ds(0, g0_len)]],
                buf[b].at[pl.ds(0, g0_len)], gsem[b])
            if g1_len:
                pltpu.async_copy(
                    table_hbm.at[idx_v.at[c, pl.ds(g0_len, g1_len)]],
                    buf[b].at[pl.ds(g0_len, g1_len)], gsem[b])

        def wait_gather(b):
            pltpu.make_async_copy(
                table_hbm.at[idx_v.at[0, pl.ds(0, seq)]], buf[b], gsem[b]).wait()

        def start_write(c, b):
            pltpu.async_copy(buf[b], out_hbm.at[base + c], wsem[b])

        def wait_write(b):
            pltpu.make_async_copy(buf[b], out_hbm.at[0], wsem[b]).wait()

        def scale(b):
            g = buf[b]

            @plsc.parallel_loop(0, seq, step=1, unroll=4)
            def _do_row(r):
                for j in range(_D // _LANES):
                    sl = pl.ds(j * _LANES, _LANES)
                    g[r, sl] = g[r, sl] * _SCALE

        start_gather(0, 0)
        start_gather(1, 1)
        for c in range(2):
            wait_gather(c)
            scale(c)
            start_write(c, c)
            start_gather(c + 2, c + 2)

        def quad_body(p, carry):
            c0 = 2 + p * _NBUF
            for k in range(_NBUF):
                b = (2 + k) % _NBUF
                c = c0 + k
                wait_gather(b)
                scale(b)
                start_write(c, b)
                wait_write((b + 2) % _NBUF)
                start_gather(c + 2, (b + 2) % _NBUF)
            return carry

        lax.fori_loop(0, (n_chunks - 4) // _NBUF, quad_body, 0)

        for c in range(n_chunks - 2, n_chunks):
            b = c % _NBUF
            wait_gather(b)
            scale(b)
            start_write(c, b)
        for b in range(_NBUF):
            wait_write(b)

    return emb


def kernel(x, lookup_table):
    batch, seq = x.shape
    return _make_kernel(batch, seq)(x.astype(jnp.int32), lookup_table)


# speedup vs baseline: 1.0150x; 1.0036x over previous
import functools
import math

import jax
import jax.numpy as jnp
from jax import lax
from jax.experimental import pallas as pl
from jax.experimental.pallas import tpu as pltpu
from jax.experimental.pallas import tpu_sc as plsc

_D = 128
_LANES = 16
_NC, _NS = 2, 16
_NW = _NC * _NS
_SCALE = math.sqrt(_D)
_NBUF = 4


def _make_kernel(batch: int, seq: int):
    rows_per_w = batch // _NW
    n_chunks = rows_per_w
    assert batch % _NW == 0 and (n_chunks - 4) % _NBUF == 0
    g0_len = min(seq, 128)
    g1_len = seq - g0_len

    mesh = plsc.VectorSubcoreMesh(
        core_axis_name="c", subcore_axis_name="s",
        num_cores=_NC, num_subcores=_NS,
    )

    @functools.partial(
        pl.kernel,
        out_type=jax.ShapeDtypeStruct((batch, seq, _D), jnp.float32),
        mesh=mesh,
        scratch_types=[
            pltpu.VMEM((rows_per_w, seq), jnp.int32),
            pltpu.VMEM((seq, _D), jnp.float32),
            pltpu.VMEM((seq, _D), jnp.float32),
            pltpu.VMEM((seq, _D), jnp.float32),
            pltpu.VMEM((seq, _D), jnp.float32),
            pltpu.SemaphoreType.DMA,
            pltpu.SemaphoreType.DMA,
            pltpu.SemaphoreType.DMA,
            pltpu.SemaphoreType.DMA,
            pltpu.SemaphoreType.DMA,
            pltpu.SemaphoreType.DMA,
            pltpu.SemaphoreType.DMA,
            pltpu.SemaphoreType.DMA,
        ],
    )
    def emb(idx_hbm, table_hbm, out_hbm, idx_v,
            b0, b1, b2, b3, gs0, gs1, gs2, gs3, ws0, ws1, ws2, ws3):
        wid = lax.axis_index("s") * _NC + lax.axis_index("c")
        base = wid * rows_per_w
        pltpu.sync_copy(idx_hbm.at[pl.ds(base, rows_per_w)], idx_v)

        buf = (b0, b1, b2, b3)
        gsem = (gs0, gs1, gs2, gs3)
        wsem = (ws0, ws1, ws2, ws3)

        def start_gather(c, b):
            pltpu.async_copy(
                table_hbm.at[idx_v.at[c, pl.ds(0, g0_len)]],
                buf[b].at[pl.ds(0, g0_len)], gsem[b])
            if g1_len:
                pltpu.async_copy(
                    table_hbm.at[idx_v.at[c, pl.ds(g0_len, g1_len)]],
                    buf[b].at[pl.ds(g0_len, g1_len)], gsem[b])

        def wait_gather(b):
            pltpu.make_async_copy(
                table_hbm.at[idx_v.at[0, pl.ds(0, seq)]], buf[b], gsem[b]).wait()

        def start_write(c, b):
            pltpu.async_copy(buf[b], out_hbm.at[base + c], wsem[b])

        def wait_write(b):
            pltpu.make_async_copy(buf[b], out_hbm.at[0], wsem[b]).wait()

        def scale(b):
            g = buf[b]

            @plsc.parallel_loop(0, seq, step=1, unroll=4)
            def _do_row(r):
                for j in range(_D // _LANES):
                    sl = pl.ds(j * _LANES, _LANES)
                    g[r, sl] = g[r, sl] * _SCALE

        start_gather(0, 0)
        start_gather(1, 1)
        for c in range(2):
            wait_gather(c)
            scale(c)
            start_write(c, c)
            start_gather(c + 2, c + 2)

        def quad_body(p, carry):
            c0 = 2 + p * _NBUF
            for k in range(_NBUF):
                b = (2 + k) % _NBUF
                c = c0 + k
                wait_gather(b)
                scale(b)
                start_write(c, b)
                wait_write((b + 2) % _NBUF)
                start_gather(c + 2, (b + 2) % _NBUF)
            return carry

        lax.fori_loop(0, (n_chunks - 4) // _NBUF, quad_body, 0)

        for c in range(n_chunks - 2, n_chunks):
            b = c % _NBUF
            wait_gather(b)
            scale(b)
            start_write(c, b)
        for b in range(_NBUF):
            wait_write(b)

    return emb


def kernel(x, lookup_table):
    batch, seq = x.shape
    if x.dtype != jnp.int32:
        x = x.astype(jnp.int32)
    return _make_kernel(batch, seq)(x, lookup_table)
